# baseline (speedup 1.0000x reference)
import functools

import jax
import jax.numpy as jnp
from jax import lax
from jax.experimental import pallas as pl
from jax.experimental.pallas import tpu as pltpu

DIM = 128
RB = 1024
UNROLL = 8


def _tc_body(idx_ref, table_ref, out_ref):
    def body(i, _):
        for u in range(UNROLL):
            r = i * UNROLL + u
            out_ref[pl.ds(r, 1), :] = table_ref[pl.ds(idx_ref[0, 0, r], 1), :]
        return ()

    lax.fori_loop(0, RB // UNROLL, body, ())


@functools.partial(jax.jit, static_argnames=("total",))
def _tc_gather(idx3, table, total):
    n_blocks = total // RB
    return pl.pallas_call(
        _tc_body,
        grid=(n_blocks,),
        in_specs=[
            pl.BlockSpec((1, 1, RB), lambda i: (i, 0, 0), memory_space=pltpu.SMEM),
            pl.BlockSpec(table.shape, lambda i: (0, 0)),
        ],
        out_specs=pl.BlockSpec((RB, DIM), lambda i: (i, 0)),
        out_shape=jax.ShapeDtypeStruct((total, DIM), jnp.float32),
    )(idx3, table)


def kernel(x, table):
    b, h = x.shape
    total = b * h
    idx3 = x.reshape(total // RB, 1, RB).astype(jnp.int32)
    out = _tc_gather(idx3, table, total)
    return out.reshape(b, h, DIM)
